# baseline (speedup 1.0000x reference)
import functools

import jax
import jax.numpy as jnp
from jax import lax
from jax.experimental import pallas as pl
from jax.experimental.pallas import tpu as pltpu
from jax.experimental.pallas import tpu_sc as plsc

NROW = 16384
NCOL = 50
D = 32
H = D // 2
NC, NS = 2, 16
NW = NC * NS
RPW = NROW // NW
R = 64
NCHUNK = RPW // R
C = R * NCOL

_mesh = plsc.VectorSubcoreMesh(core_axis_name="c", subcore_axis_name="s")


@functools.partial(
    pl.kernel,
    out_type=jax.ShapeDtypeStruct((NROW, 56, 128), jnp.float32),
    mesh=_mesh,
    scratch_types=[
        pltpu.VMEM((R, NCOL), jnp.int32),
        pltpu.VMEM((C,), jnp.int32),
        pltpu.VMEM((C, H), jnp.float32),
        pltpu.VMEM((C, H), jnp.float32),
        pltpu.SemaphoreType.DMA,
        pltpu.SemaphoreType.DMA,
    ],
    compiler_params=pltpu.CompilerParams(
        use_tc_tiling_on_sc=False, needs_layout_passes=False),
)
def _gather_kernel(x_hbm, ta_hbm, tb_hbm, out_hbm, xchunk_v, idx_v,
                   rows_a, rows_b, sem, sem_wb):
    wid = lax.axis_index("s") * NC + lax.axis_index("c")
    base_row = wid * RPW
    lanes = lax.iota(jnp.int32, 16)

    for c in range(NCHUNK):
        r0 = base_row + c * R
        pltpu.sync_copy(x_hbm.at[pl.ds(r0, R), :], xchunk_v)

        def transpose_step(t, _):
            k = t * 16 + lanes
            r = jnp.bitwise_and(k, R - 1)
            j = jnp.right_shift(k, 6)
            vals = plsc.load_gather(xchunk_v, [r, j])
            idx_v[pl.ds(t * 16, 16)] = vals
            return _

        lax.fori_loop(0, C // 16, transpose_step, 0)

        ga = pltpu.async_copy(ta_hbm.at[idx_v], rows_a, sem)
        gb = pltpu.async_copy(tb_hbm.at[idx_v], rows_b, sem)
        ga.wait()
        gb.wait()

        wb = [
            pltpu.async_copy(
                rows.at[pl.ds(j * R, R), :],
                out_hbm.at[pl.ds(r0, R), j, pl.ds(off, H)], sem_wb)
            for j in range(NCOL)
            for rows, off in ((rows_a, 0), (rows_b, H))
        ]
        for d in wb:
            d.wait()


def kernel(x, embedding):
    out_padded = _gather_kernel(x, embedding[:, :H], embedding[:, H:])
    return out_padded[:, :NCOL, :D]
